# baseline (speedup 1.0000x reference)
import functools

import jax
import jax.numpy as jnp
from jax import lax
from jax.experimental import pallas as pl
from jax.experimental.pallas import tpu as pltpu
from jax.experimental.pallas import tpu_sc as plsc


def _make_sc_gather(V, D, B, NC, NS):
    NW = NC * NS
    b_per_w = B // NW
    mesh = plsc.VectorSubcoreMesh(core_axis_name="c", subcore_axis_name="s")

    @functools.partial(
        pl.kernel,
        out_type=jax.ShapeDtypeStruct((B, D), jnp.float32),
        mesh=mesh,
        scratch_types=[
            pltpu.VMEM((b_per_w,), jnp.int32),
            pltpu.VMEM((b_per_w, D), jnp.float32),
            pltpu.SemaphoreType.DMA,
        ],
        compiler_params=pltpu.CompilerParams(use_tc_tiling_on_sc=False),
    )
    def gather_k(table_hbm, idx_hbm, out_hbm, idx_v, rows_v, sem):
        wid = lax.axis_index("s") * NC + lax.axis_index("c")
        base = wid * b_per_w
        pltpu.sync_copy(idx_hbm.at[pl.ds(base, b_per_w)], idx_v)
        pltpu.async_copy(table_hbm.at[idx_v], rows_v, sem).wait()
        pltpu.sync_copy(rows_v, out_hbm.at[pl.ds(base, b_per_w)])

    return gather_k


def _matmul_body(x_ref, pos_ref, w_ref, out_ref):
    x = x_ref[...] + pos_ref[...]
    out_ref[...] = lax.dot_general(
        x, w_ref[...],
        dimension_numbers=(((1,), (1,)), ((), ())),
        preferred_element_type=jnp.float32)


def kernel(in_idx, tok_emb, pos_emb, W_out):
    B, S = in_idx.shape
    V, D = tok_emb.shape
    flat_idx = in_idx.reshape(B * S).astype(jnp.int32)

    info = plsc.get_sparse_core_info()
    gather = _make_sc_gather(V, D, B * S, info.num_cores, info.num_subcores)
    xg = gather(tok_emb, flat_idx)

    MT = S
    VT = 1024
    logits = pl.pallas_call(
        _matmul_body,
        grid=(B * S // MT, pl.cdiv(V, VT)),
        in_specs=[
            pl.BlockSpec((MT, D), lambda i, j: (i, 0)),
            pl.BlockSpec((S, D), lambda i, j: (0, 0)),
            pl.BlockSpec((VT, D), lambda i, j: (j, 0)),
        ],
        out_specs=pl.BlockSpec((MT, VT), lambda i, j: (i, j)),
        out_shape=jax.ShapeDtypeStruct((B * S, V), jnp.float32),
        compiler_params=pltpu.CompilerParams(
            dimension_semantics=("parallel", "arbitrary")),
    )(xg, pos_emb, W_out)
    return logits.reshape(B, S, V)


# speedup vs baseline: 1.1037x; 1.1037x over previous
import functools

import jax
import jax.numpy as jnp
from jax import lax
from jax.experimental import pallas as pl
from jax.experimental.pallas import tpu as pltpu
from jax.experimental.pallas import tpu_sc as plsc


def _make_sc_gather(V, DP, B, NC, NS):
    NW = NC * NS
    b_per_w = B // NW
    mesh = plsc.VectorSubcoreMesh(core_axis_name="c", subcore_axis_name="s")

    @functools.partial(
        pl.kernel,
        out_type=jax.ShapeDtypeStruct((B, DP), jnp.float32),
        mesh=mesh,
        scratch_types=[
            pltpu.VMEM((b_per_w,), jnp.int32),
            pltpu.VMEM((b_per_w, DP), jnp.float32),
            pltpu.SemaphoreType.DMA,
        ],
    )
    def gather_k(table_hbm, idx_hbm, out_hbm, idx_v, rows_v, sem):
        wid = lax.axis_index("s") * NC + lax.axis_index("c")
        base = wid * b_per_w
        pltpu.sync_copy(idx_hbm.at[pl.ds(base, b_per_w)], idx_v)
        pltpu.async_copy(table_hbm.at[idx_v], rows_v, sem).wait()
        pltpu.sync_copy(rows_v, out_hbm.at[pl.ds(base, b_per_w)])

    return gather_k


def _matmul_body(x_ref, pos_ref, w_ref, out_ref):
    x = x_ref[...][:, :64] + pos_ref[...]
    out_ref[...] = lax.dot_general(
        x, w_ref[...],
        dimension_numbers=(((1,), (1,)), ((), ())),
        preferred_element_type=jnp.float32)


def kernel(in_idx, tok_emb, pos_emb, W_out):
    B, S = in_idx.shape
    V, D = tok_emb.shape
    DP = 128
    flat_idx = in_idx.reshape(B * S).astype(jnp.int32)
    tok_pad = jnp.pad(tok_emb, ((0, 0), (0, DP - D)))

    info = plsc.get_sparse_core_info()
    gather = _make_sc_gather(V, DP, B * S, info.num_cores, info.num_subcores)
    xg = gather(tok_pad, flat_idx)

    M = B * S
    VT = 1024
    pos_full = jnp.tile(pos_emb, (B, 1))
    logits = pl.pallas_call(
        _matmul_body,
        grid=(pl.cdiv(V, VT),),
        in_specs=[
            pl.BlockSpec((M, DP), lambda j: (0, 0)),
            pl.BlockSpec((M, D), lambda j: (0, 0)),
            pl.BlockSpec((VT, D), lambda j: (j, 0)),
        ],
        out_specs=pl.BlockSpec((M, VT), lambda j: (0, j)),
        out_shape=jax.ShapeDtypeStruct((M, V), jnp.float32),
        compiler_params=pltpu.CompilerParams(
            dimension_semantics=("arbitrary",)),
    )(xg, pos_full, W_out)
    return logits.reshape(B, S, V)
